# baseline (speedup 1.0000x reference)
import functools

import jax
import jax.numpy as jnp
from jax import lax
from jax.experimental import pallas as pl
from jax.experimental.pallas import tpu as pltpu
from jax.experimental.pallas import tpu_sc as plsc

_N = 1024
_L = 20
_NW = 32
_PAIRS = _L * 27
_PPW = 17
_WIN = 12


def _sc_body(item_hbm, hist_hbm, ctx_hbm, out_hbm,
             item_s, hist_s, ctx_s, sem_in, sem_out):
    cid = lax.axis_index("c")
    sid = lax.axis_index("s")
    wid = cid * 16 + sid

    @pl.when(sid < 15)
    def _():
        r0 = sid * 64
        pltpu.async_copy(hist_hbm.at[pl.ds(r0, 64)],
                         hist_s.at[pl.ds(r0, 64)], sem_in)
        pltpu.make_async_copy(hist_hbm.at[pl.ds(r0, 64)],
                              hist_s.at[pl.ds(r0, 64)], sem_in).wait()

    @pl.when(sid == 15)
    def _():
        pltpu.async_copy(item_hbm, item_s, sem_in)
        pltpu.async_copy(ctx_hbm, ctx_s, sem_in)
        pltpu.make_async_copy(item_hbm, item_s, sem_in).wait()
        pltpu.make_async_copy(ctx_hbm, ctx_s, sem_in).wait()

    plsc.subcore_barrier()

    start = wid * _PPW - lax.max(wid - 28, 0)
    count = jnp.where(wid < 28, _PPW, _PPW - 1)

    def copies(p):
        l = p // 27
        g = p % 27
        i = g // 9
        j = (g % 9) // 3
        k = g % 3
        r0 = l * 1296 + g * 48
        return (
            pltpu.make_async_copy(item_s.at[pl.ds(i * 16, 16)],
                                  out_hbm.at[pl.ds(r0, 16)], sem_out),
            pltpu.make_async_copy(ctx_s.at[pl.ds(k * 16, 16)],
                                  out_hbm.at[pl.ds(r0 + 16, 16)], sem_out),
            pltpu.make_async_copy(hist_s.at[pl.ds(l * 48 + j * 16, 16)],
                                  out_hbm.at[pl.ds(r0 + 32, 16)], sem_out),
        )

    def drain_one():
        d = pltpu.make_async_copy(item_s.at[pl.ds(0, 16)],
                                  out_hbm.at[pl.ds(0, 16)], sem_out)
        d.wait()
        d.wait()
        d.wait()

    def body(it, carry):
        @pl.when(it < count)
        def _():
            a, b, c = copies(start + it)
            a.start()
            b.start()
            c.start()

        @pl.when((it >= _WIN) & (it - _WIN < count))
        def _():
            drain_one()

        return carry

    lax.fori_loop(0, _PPW, body, 0)

    rem = count - (_PPW - _WIN)

    def tail(it, carry):
        @pl.when(it < rem)
        def _():
            drain_one()

        return carry

    lax.fori_loop(0, _WIN, tail, 0)


def kernel(x_item, hist, x_context):
    mesh = plsc.VectorSubcoreMesh(core_axis_name="c", subcore_axis_name="s")
    run = functools.partial(
        pl.kernel,
        _sc_body,
        mesh=mesh,
        out_type=jax.ShapeDtypeStruct((_PAIRS * 48, _N), jnp.float32),
        scratch_types=[
            pltpu.VMEM_SHARED((48, _N), jnp.float32),
            pltpu.VMEM_SHARED((960, _N), jnp.float32),
            pltpu.VMEM_SHARED((48, _N), jnp.float32),
            pltpu.SemaphoreType.DMA,
            pltpu.SemaphoreType.DMA,
        ],
    )()
    flat = run(x_item.reshape(_N, 48).T, hist.reshape(_N, 960).T,
               x_context.reshape(_N, 48).T)
    return jnp.swapaxes(flat, 0, 1).reshape(_N, _L, 27, 48)


# speedup vs baseline: 1.0158x; 1.0158x over previous
import functools

import jax
import jax.numpy as jnp
from jax import lax
from jax.experimental import pallas as pl
from jax.experimental.pallas import tpu as pltpu
from jax.experimental.pallas import tpu_sc as plsc

_N = 1024
_L = 20
_NW = 32
_PAIRS = _L * 27
_PPW = 17
_WIN = 6


def _sc_body(item_hbm, hist_hbm, ctx_hbm, out_hbm,
             item_s, hist_s, ctx_s, sem_in, sem_out):
    cid = lax.axis_index("c")
    sid = lax.axis_index("s")
    wid = cid * 16 + sid

    @pl.when(sid < 15)
    def _():
        r0 = sid * 64
        pltpu.async_copy(hist_hbm.at[pl.ds(r0, 64)],
                         hist_s.at[pl.ds(r0, 64)], sem_in)
        pltpu.make_async_copy(hist_hbm.at[pl.ds(r0, 64)],
                              hist_s.at[pl.ds(r0, 64)], sem_in).wait()

    @pl.when(sid == 15)
    def _():
        pltpu.async_copy(item_hbm, item_s, sem_in)
        pltpu.async_copy(ctx_hbm, ctx_s, sem_in)
        pltpu.make_async_copy(item_hbm, item_s, sem_in).wait()
        pltpu.make_async_copy(ctx_hbm, ctx_s, sem_in).wait()

    plsc.subcore_barrier()

    start = wid * _PPW - lax.max(wid - 28, 0)
    count = jnp.where(wid < 28, _PPW, _PPW - 1)

    def copies(p):
        l = p // 27
        g = p % 27
        i = g // 9
        j = (g % 9) // 3
        k = g % 3
        r0 = l * 1296 + g * 48
        return (
            pltpu.make_async_copy(item_s.at[pl.ds(i * 16, 16)],
                                  out_hbm.at[pl.ds(r0, 16)], sem_out),
            pltpu.make_async_copy(ctx_s.at[pl.ds(k * 16, 16)],
                                  out_hbm.at[pl.ds(r0 + 16, 16)], sem_out),
            pltpu.make_async_copy(hist_s.at[pl.ds(l * 48 + j * 16, 16)],
                                  out_hbm.at[pl.ds(r0 + 32, 16)], sem_out),
        )

    def drain_one():
        d = pltpu.make_async_copy(item_s.at[pl.ds(0, 16)],
                                  out_hbm.at[pl.ds(0, 16)], sem_out)
        d.wait()
        d.wait()
        d.wait()

    def body(it, carry):
        @pl.when(it < count)
        def _():
            a, b, c = copies(start + it)
            a.start()
            b.start()
            c.start()

        @pl.when((it >= _WIN) & (it - _WIN < count))
        def _():
            drain_one()

        return carry

    lax.fori_loop(0, _PPW, body, 0)

    rem = count - (_PPW - _WIN)

    def tail(it, carry):
        @pl.when(it < rem)
        def _():
            drain_one()

        return carry

    lax.fori_loop(0, _WIN, tail, 0)


def kernel(x_item, hist, x_context):
    mesh = plsc.VectorSubcoreMesh(core_axis_name="c", subcore_axis_name="s")
    run = functools.partial(
        pl.kernel,
        _sc_body,
        mesh=mesh,
        out_type=jax.ShapeDtypeStruct((_PAIRS * 48, _N), jnp.float32),
        scratch_types=[
            pltpu.VMEM_SHARED((48, _N), jnp.float32),
            pltpu.VMEM_SHARED((960, _N), jnp.float32),
            pltpu.VMEM_SHARED((48, _N), jnp.float32),
            pltpu.SemaphoreType.DMA,
            pltpu.SemaphoreType.DMA,
        ],
    )()
    flat = run(x_item.reshape(_N, 48).T, hist.reshape(_N, 960).T,
               x_context.reshape(_N, 48).T)
    return jnp.swapaxes(flat, 0, 1).reshape(_N, _L, 27, 48)
